# speedup vs baseline: 1.0000x; 1.0000x over previous
import math

import jax
import jax.numpy as jnp
from jax.experimental import pallas as pl
from jax.experimental.pallas import tpu as pltpu

_INPUT_SIZE = 31 * 1024
_H1 = 512
_H2 = 256
_EMBED = 77 * 1024

_P1_TK = 3968
_P1_TN = 256
_P3_TN = 5632

_VMEM_LIMIT = 96 * 1024 * 1024


def _gelu(x):
    return 0.5 * x * (1.0 + jax.lax.erf(x * (1.0 / math.sqrt(2.0))))


def _proj1_kernel(x_ref, w_ref, b_ref, o_ref, acc_ref):
    k = pl.program_id(1)

    @pl.when(k == 0)
    def _():
        acc_ref[...] = jnp.zeros_like(acc_ref)

    acc_ref[...] += jnp.dot(
        x_ref[...].astype(jnp.bfloat16),
        w_ref[...].astype(jnp.bfloat16),
        preferred_element_type=jnp.float32,
    )

    @pl.when(k == pl.num_programs(1) - 1)
    def _():
        o_ref[...] = _gelu(acc_ref[...] + b_ref[...]).astype(o_ref.dtype)


def _proj1(x, w1, b1):
    M = x.shape[0]
    nk = _INPUT_SIZE // _P1_TK
    nj = _H1 // _P1_TN
    return pl.pallas_call(
        _proj1_kernel,
        out_shape=jax.ShapeDtypeStruct((M, _H1), jnp.bfloat16),
        grid_spec=pltpu.PrefetchScalarGridSpec(
            num_scalar_prefetch=0,
            grid=(nj, nk),
            in_specs=[
                pl.BlockSpec((M, _P1_TK), lambda j, k: (0, k)),
                pl.BlockSpec((_P1_TK, _P1_TN), lambda j, k: (k, j)),
                pl.BlockSpec((1, _P1_TN), lambda j, k: (0, j)),
            ],
            out_specs=pl.BlockSpec((M, _P1_TN), lambda j, k: (0, j)),
            scratch_shapes=[pltpu.VMEM((M, _P1_TN), jnp.float32)],
        ),
        compiler_params=pltpu.CompilerParams(
            dimension_semantics=("parallel", "arbitrary"),
            vmem_limit_bytes=_VMEM_LIMIT,
        ),
    )(x, w1, b1.reshape(1, _H1))


def _proj23_kernel(h1_ref, w2_ref, b2_ref, w3_ref, b3_ref, o_ref):
    h2 = _gelu(
        jnp.dot(h1_ref[...], w2_ref[...], preferred_element_type=jnp.float32)
        + b2_ref[...]
    )
    out = jnp.dot(
        h2.astype(jnp.bfloat16),
        w3_ref[...].astype(jnp.bfloat16),
        preferred_element_type=jnp.float32,
    ) + b3_ref[...]
    o_ref[...] = out


def _proj23(h1, w2, b2, w3, b3):
    M = h1.shape[0]
    nj = _EMBED // _P3_TN
    return pl.pallas_call(
        _proj23_kernel,
        out_shape=jax.ShapeDtypeStruct((M, _EMBED), jnp.float32),
        grid_spec=pltpu.PrefetchScalarGridSpec(
            num_scalar_prefetch=0,
            grid=(nj,),
            in_specs=[
                pl.BlockSpec((M, _H1), lambda j: (0, 0)),
                pl.BlockSpec((_H1, _H2), lambda j: (0, 0)),
                pl.BlockSpec((1, _H2), lambda j: (0, 0)),
                pl.BlockSpec((_H2, _P3_TN), lambda j: (0, j)),
                pl.BlockSpec((1, _P3_TN), lambda j: (0, j)),
            ],
            out_specs=pl.BlockSpec((M, _P3_TN), lambda j: (0, j)),
        ),
        compiler_params=pltpu.CompilerParams(
            dimension_semantics=("parallel",),
            vmem_limit_bytes=_VMEM_LIMIT,
        ),
    )(h1, w2, b2.reshape(1, _H2), w3, b3.reshape(1, _EMBED))


@jax.jit
def kernel(x, w1, b1, w2, b2, w3, b3):
    batch = x.shape[0]
    x = x.reshape(batch, -1)
    h1 = _proj1(x, w1, b1)
    out = _proj23(h1, w2.astype(jnp.bfloat16), b2, w3, b3)
    return out.reshape(batch, 77, 1024)


# speedup vs baseline: 1.2826x; 1.2826x over previous
import math

import jax
import jax.numpy as jnp
from jax.experimental import pallas as pl
from jax.experimental.pallas import tpu as pltpu

_K_CHUNKS = 31
_H1 = 512
_H2 = 256
_N_CHUNKS = 77

_P1_TI = 8
_P3_TI = 8

_VMEM_LIMIT = 100 * 1024 * 1024


def _gelu(x):
    return 0.5 * x * (1.0 + jax.lax.erf(x * (1.0 / math.sqrt(2.0))))


def _proj1_kernel(x_hbm, w_ref, b_ref, o_ref, acc_ref, xbuf, sem):
    k = pl.program_id(1)
    nk = pl.num_programs(1)

    def _copy(chunk, slot):
        return pltpu.make_async_copy(
            x_hbm.at[:, chunk, :], xbuf.at[slot], sem.at[slot])

    @pl.when(k == 0)
    def _():
        acc_ref[...] = jnp.zeros_like(acc_ref)
        _copy(0, 0).start()

    @pl.when(k + 1 < nk)
    def _():
        _copy(k + 1, (k + 1) % 2).start()

    _copy(k, k % 2).wait()
    xi = xbuf[k % 2].astype(jnp.bfloat16)
    acc_ref[...] += jnp.dot(xi, w_ref[...].astype(jnp.bfloat16),
                            preferred_element_type=jnp.float32)

    @pl.when(k == nk - 1)
    def _():
        o_ref[...] = _gelu(acc_ref[...] + b_ref[...]).astype(o_ref.dtype)


def _proj1(x, w1, b1):
    M = x.shape[0]
    nk = _K_CHUNKS
    nj = 2
    tn = _H1 // nj
    return pl.pallas_call(
        _proj1_kernel,
        out_shape=jax.ShapeDtypeStruct((M, _H1), jnp.bfloat16),
        grid_spec=pltpu.PrefetchScalarGridSpec(
            num_scalar_prefetch=0,
            grid=(nj, nk),
            in_specs=[
                pl.BlockSpec(memory_space=pl.ANY),
                pl.BlockSpec((1024, tn), lambda j, k: (k, j)),
                pl.BlockSpec((1, tn), lambda j, k: (0, j)),
            ],
            out_specs=pl.BlockSpec((M, tn), lambda j, k: (0, j)),
            scratch_shapes=[
                pltpu.VMEM((M, tn), jnp.float32),
                pltpu.VMEM((2, M, 1024), jnp.float32),
                pltpu.SemaphoreType.DMA((2,)),
            ],
        ),
        compiler_params=pltpu.CompilerParams(
            dimension_semantics=("parallel", "arbitrary"),
            vmem_limit_bytes=_VMEM_LIMIT,
        ),
    )(x, w1, b1.reshape(1, _H1))


def _proj23_kernel(h1_ref, w2_ref, b2_ref, w3_ref, b3_ref, o_ref):
    h2 = _gelu(
        jnp.dot(h1_ref[...], w2_ref[...], preferred_element_type=jnp.float32)
        + b2_ref[...]
    ).astype(jnp.bfloat16)
    w3_all = w3_ref[...]
    b3_all = b3_ref[...]
    for i in range(_P3_TI):
        wv = w3_all[:, 1024 * i:1024 * (i + 1)].astype(jnp.bfloat16)
        val = jnp.dot(h2, wv, preferred_element_type=jnp.float32)
        o_ref[:, i, :] = val + b3_all[:, 1024 * i:1024 * (i + 1)]


def _proj23(h1, w2, b2, w3, b3):
    M = h1.shape[0]
    nj = (_N_CHUNKS + _P3_TI - 1) // _P3_TI
    tn = _P3_TI * 1024
    return pl.pallas_call(
        _proj23_kernel,
        out_shape=jax.ShapeDtypeStruct((M, _N_CHUNKS, 1024), jnp.float32),
        grid_spec=pltpu.PrefetchScalarGridSpec(
            num_scalar_prefetch=0,
            grid=(nj,),
            in_specs=[
                pl.BlockSpec((M, _H1), lambda j: (0, 0)),
                pl.BlockSpec((_H1, _H2), lambda j: (0, 0)),
                pl.BlockSpec((1, _H2), lambda j: (0, 0)),
                pl.BlockSpec((_H2, tn), lambda j: (0, j)),
                pl.BlockSpec((1, tn), lambda j: (0, j)),
            ],
            out_specs=pl.BlockSpec((M, _P3_TI, 1024), lambda j: (0, j, 0)),
        ),
        compiler_params=pltpu.CompilerParams(
            dimension_semantics=("parallel",),
            vmem_limit_bytes=_VMEM_LIMIT,
        ),
    )(h1, w2, b2.reshape(1, _H2), w3, b3.reshape(1, _N_CHUNKS * 1024))


@jax.jit
def kernel(x, w1, b1, w2, b2, w3, b3):
    h1 = _proj1(x, w1, b1)
    return _proj23(h1, w2.astype(jnp.bfloat16), b2, w3, b3)


# speedup vs baseline: 3.1328x; 2.4425x over previous
import math

import jax
import jax.numpy as jnp
from jax.experimental import pallas as pl
from jax.experimental.pallas import tpu as pltpu

_K_CHUNKS = 31
_H1 = 512
_H2 = 256
_N_CHUNKS = 77

_P1_TI = 4
_P3_TI = 8

_VMEM_LIMIT = 100 * 1024 * 1024


def _gelu(x):
    return 0.5 * x * (1.0 + jax.lax.erf(x * (1.0 / math.sqrt(2.0))))


def _proj1_kernel(x_ref, w_ref, b_ref, o_ref, acc_ref):
    k = pl.program_id(1)
    nk = pl.num_programs(1)

    @pl.when(k == 0)
    def _():
        acc_ref[...] = jnp.zeros_like(acc_ref)

    w_all = w_ref[...]
    acc = acc_ref[...]
    for i in range(_P1_TI):
        valid = (k * _P1_TI + i) < _K_CHUNKS
        xi = jnp.where(valid, x_ref[i], 0.0).astype(jnp.bfloat16)
        wi = jnp.where(valid, w_all[1024 * i:1024 * (i + 1), :], 0.0)
        acc += jnp.dot(xi, wi.astype(jnp.bfloat16),
                       preferred_element_type=jnp.float32)
    acc_ref[...] = acc

    @pl.when(k == nk - 1)
    def _():
        o_ref[...] = _gelu(acc + b_ref[...]).astype(o_ref.dtype)


def _proj1(xt, w1, b1):
    M = xt.shape[1]
    nk = (_K_CHUNKS + _P1_TI - 1) // _P1_TI
    nj = 2
    tn = _H1 // nj
    return pl.pallas_call(
        _proj1_kernel,
        out_shape=jax.ShapeDtypeStruct((M, _H1), jnp.bfloat16),
        grid_spec=pltpu.PrefetchScalarGridSpec(
            num_scalar_prefetch=0,
            grid=(nj, nk),
            in_specs=[
                pl.BlockSpec((_P1_TI, M, 1024), lambda j, k: (k, 0, 0)),
                pl.BlockSpec((_P1_TI * 1024, tn), lambda j, k: (k, j)),
                pl.BlockSpec((1, tn), lambda j, k: (0, j)),
            ],
            out_specs=pl.BlockSpec((M, tn), lambda j, k: (0, j)),
            scratch_shapes=[pltpu.VMEM((M, tn), jnp.float32)],
        ),
        compiler_params=pltpu.CompilerParams(
            dimension_semantics=("parallel", "arbitrary"),
            vmem_limit_bytes=_VMEM_LIMIT,
        ),
    )(xt, w1, b1.reshape(1, _H1))


def _proj23_kernel(h1_ref, w2_ref, b2_ref, w3_ref, b3_ref, o_ref):
    h2 = _gelu(
        jnp.dot(h1_ref[...], w2_ref[...], preferred_element_type=jnp.float32)
        + b2_ref[...]
    ).astype(jnp.bfloat16)
    w3_all = w3_ref[...]
    b3_all = b3_ref[...]
    for i in range(_P3_TI):
        wv = w3_all[:, 1024 * i:1024 * (i + 1)].astype(jnp.bfloat16)
        val = jnp.dot(h2, wv, preferred_element_type=jnp.float32)
        o_ref[i] = val + b3_all[:, 1024 * i:1024 * (i + 1)]


def _proj23(h1, w2, b2, w3, b3):
    M = h1.shape[0]
    nj = (_N_CHUNKS + _P3_TI - 1) // _P3_TI
    tn = _P3_TI * 1024
    return pl.pallas_call(
        _proj23_kernel,
        out_shape=jax.ShapeDtypeStruct((_N_CHUNKS, M, 1024), jnp.float32),
        grid_spec=pltpu.PrefetchScalarGridSpec(
            num_scalar_prefetch=0,
            grid=(nj,),
            in_specs=[
                pl.BlockSpec((M, _H1), lambda j: (0, 0)),
                pl.BlockSpec((_H1, _H2), lambda j: (0, 0)),
                pl.BlockSpec((1, _H2), lambda j: (0, 0)),
                pl.BlockSpec((_H2, tn), lambda j: (0, j)),
                pl.BlockSpec((1, tn), lambda j: (0, j)),
            ],
            out_specs=pl.BlockSpec((_P3_TI, M, 1024), lambda j: (j, 0, 0)),
        ),
        compiler_params=pltpu.CompilerParams(
            dimension_semantics=("parallel",),
            vmem_limit_bytes=_VMEM_LIMIT,
        ),
    )(h1, w2, b2.reshape(1, _H2), w3, b3.reshape(1, _N_CHUNKS * 1024))


@jax.jit
def kernel(x, w1, b1, w2, b2, w3, b3):
    xt = jnp.transpose(x, (1, 0, 2))
    h1 = _proj1(xt, w1, b1)
    out = _proj23(h1, w2.astype(jnp.bfloat16), b2, w3, b3)
    return jnp.transpose(out, (1, 0, 2))


# speedup vs baseline: 3.5048x; 1.1188x over previous
import math

import jax
import jax.numpy as jnp
from jax.experimental import pallas as pl
from jax.experimental.pallas import tpu as pltpu

_K_CHUNKS = 31
_H1 = 512
_H2 = 256
_N_CHUNKS = 77

_P1_TI = 4
_P1_STEPS = 4
_P3_TI = 8

_VMEM_LIMIT = 100 * 1024 * 1024


def _gelu(x):
    return 0.5 * x * (1.0 + jax.lax.erf(x * (1.0 / math.sqrt(2.0))))


def _proj1_kernel(x_ref, w_ref, o_ref, acc_ref):
    kh = pl.program_id(0)
    k = pl.program_id(1)
    nk = pl.num_programs(1)

    @pl.when(k == 0)
    def _():
        acc_ref[...] = jnp.zeros_like(acc_ref)

    w_all = w_ref[...]
    acc = acc_ref[...]
    base = (kh * nk + k) * _P1_TI
    for i in range(_P1_TI):
        valid = base + i < _K_CHUNKS
        xi = jnp.where(valid, x_ref[i], 0.0).astype(jnp.bfloat16)
        wi = jnp.where(valid, w_all[1024 * i:1024 * (i + 1), :], 0.0)
        acc += jnp.dot(xi, wi.astype(jnp.bfloat16),
                       preferred_element_type=jnp.float32)
    acc_ref[...] = acc

    @pl.when(k == nk - 1)
    def _():
        o_ref[0] = acc


def _proj1(xt, w1):
    M = xt.shape[1]
    return pl.pallas_call(
        _proj1_kernel,
        out_shape=jax.ShapeDtypeStruct((2, M, _H1), jnp.float32),
        grid_spec=pltpu.PrefetchScalarGridSpec(
            num_scalar_prefetch=0,
            grid=(2, _P1_STEPS),
            in_specs=[
                pl.BlockSpec((_P1_TI, M, 1024),
                             lambda kh, k: (kh * _P1_STEPS + k, 0, 0)),
                pl.BlockSpec((_P1_TI * 1024, _H1),
                             lambda kh, k: (kh * _P1_STEPS + k, 0)),
            ],
            out_specs=pl.BlockSpec((1, M, _H1), lambda kh, k: (kh, 0, 0)),
            scratch_shapes=[pltpu.VMEM((M, _H1), jnp.float32)],
        ),
        compiler_params=pltpu.CompilerParams(
            dimension_semantics=("parallel", "arbitrary"),
            vmem_limit_bytes=_VMEM_LIMIT,
        ),
    )(xt, w1)


def _proj23_kernel(h1p_ref, b1_ref, w2_ref, b2_ref, w3_ref, b3_ref, o_ref):
    h1 = _gelu(h1p_ref[0] + h1p_ref[1] + b1_ref[...]).astype(jnp.bfloat16)
    h2 = _gelu(
        jnp.dot(h1, w2_ref[...].astype(jnp.bfloat16),
                preferred_element_type=jnp.float32)
        + b2_ref[...]
    ).astype(jnp.bfloat16)
    w3_all = w3_ref[...]
    b3_all = b3_ref[...]
    for i in range(_P3_TI):
        wv = w3_all[:, 1024 * i:1024 * (i + 1)].astype(jnp.bfloat16)
        val = jnp.dot(h2, wv, preferred_element_type=jnp.float32)
        o_ref[i] = val + b3_all[:, 1024 * i:1024 * (i + 1)]


def _proj23(h1p, b1, w2, b2, w3, b3):
    M = h1p.shape[1]
    nj = (_N_CHUNKS + _P3_TI - 1) // _P3_TI
    tn = _P3_TI * 1024
    return pl.pallas_call(
        _proj23_kernel,
        out_shape=jax.ShapeDtypeStruct((_N_CHUNKS, M, 1024), jnp.float32),
        grid_spec=pltpu.PrefetchScalarGridSpec(
            num_scalar_prefetch=0,
            grid=(nj,),
            in_specs=[
                pl.BlockSpec((2, M, _H1), lambda j: (0, 0, 0)),
                pl.BlockSpec((1, _H1), lambda j: (0, 0)),
                pl.BlockSpec((_H1, _H2), lambda j: (0, 0)),
                pl.BlockSpec((1, _H2), lambda j: (0, 0)),
                pl.BlockSpec((_H2, tn), lambda j: (0, j)),
                pl.BlockSpec((1, tn), lambda j: (0, j)),
            ],
            out_specs=pl.BlockSpec((_P3_TI, M, 1024), lambda j: (j, 0, 0)),
        ),
        compiler_params=pltpu.CompilerParams(
            dimension_semantics=("parallel",),
            vmem_limit_bytes=_VMEM_LIMIT,
        ),
    )(h1p, b1.reshape(1, _H1), w2, b2.reshape(1, _H2), w3,
      b3.reshape(1, _N_CHUNKS * 1024))


@jax.jit
def kernel(x, w1, b1, w2, b2, w3, b3):
    xt = jnp.transpose(x, (1, 0, 2))
    h1p = _proj1(xt, w1)
    out = _proj23(h1p, b1, w2, b2, w3, b3)
    return jnp.transpose(out, (1, 0, 2))
